# baseline (speedup 1.0000x reference)
import functools

import jax
import jax.numpy as jnp
from jax import lax
from jax.experimental import pallas as pl
from jax.experimental.pallas import tpu as pltpu
from jax.experimental.pallas import tpu_sc as plsc


@functools.lru_cache(maxsize=None)
def _build_warp(B, C, H, W):
    HW = H * W
    N = B * HW
    CH = C // 2
    info = plsc.get_sparse_core_info()
    NC = info.num_cores
    NW = NC * info.num_subcores
    L = info.num_lanes
    K = 64
    SLOTS = 4
    assert N % NW == 0 and C % (2 * L) == 0
    PPW = N // NW
    assert PPW % (K * SLOTS) == 0
    NCHUNK = PPW // K
    T = NCHUNK // SLOTS
    HALF = NCHUNK // 2
    RELOAD_T = (HALF - SLOTS) // SLOTS
    PPW2 = PPW // 2
    GPC = K // L
    LOG_HW = HW.bit_length() - 1
    assert (1 << LOG_HW) == HW and (1 << (W.bit_length() - 1)) == W

    mesh = plsc.VectorSubcoreMesh(core_axis_name="core", subcore_axis_name="sub")

    scratch = (
        [pltpu.VMEM((PPW2,), jnp.float32)] * 2
        + [pltpu.VMEM((4, K), jnp.int32)] * SLOTS
        + [pltpu.VMEM((4, K), jnp.float32)] * SLOTS
        + [pltpu.VMEM((4, K, CH), jnp.int32)] * SLOTS
        + [pltpu.VMEM((CH, K), jnp.int32)] * SLOTS
        + [pltpu.SemaphoreType.DMA] * (2 * SLOTS)
    )

    @functools.partial(
        pl.kernel,
        out_type=jax.ShapeDtypeStruct((B, CH, HW), jnp.int32),
        mesh=mesh,
        compiler_params=pltpu.CompilerParams(
            use_tc_tiling_on_sc=False, needs_layout_passes=False),
        scratch_types=scratch,
    )
    def warp(table, flow_hbm, out_hbm, *s):
        fxv, fyv = s[0], s[1]
        idxs = s[2:2 + SLOTS]
        wgts = s[2 + SLOTS:2 + 2 * SLOTS]
        rs = s[2 + 2 * SLOTS:2 + 3 * SLOTS]
        outs = s[2 + 3 * SLOTS:2 + 4 * SLOTS]
        semg = s[2 + 4 * SLOTS:2 + 5 * SLOTS]
        semw = s[2 + 5 * SLOTS:2 + 6 * SLOTS]
        wid = lax.axis_index("sub") * NC + lax.axis_index("core")
        base = wid * PPW
        lanes = lax.iota(jnp.int32, L)

        wb = base >> LOG_HW
        wyx = pl.multiple_of(base - (wb << LOG_HW), K)

        def load_flow(halfsel):
            off = pl.multiple_of(wyx + halfsel * PPW2, K)
            pltpu.sync_copy(flow_hbm.at[2 * wb, pl.ds(off, PPW2)], fxv)
            pltpu.sync_copy(flow_hbm.at[2 * wb + 1, pl.ds(off, PPW2)], fyv)

        def phase1(ci, idxr, wgtr):
            p0 = base + ci * K
            f0 = (ci & (HALF - 1)) * K
            for g in range(GPC):
                sg = g * L
                sl = pl.ds(sg, L)
                p = p0 + sg + lanes
                x = p & (W - 1)
                y = (p >> (W.bit_length() - 1)) & (H - 1)
                boff = p - (p & (HW - 1))
                xf = x.astype(jnp.float32) + fxv[pl.ds(f0 + sg, L)]
                yf = y.astype(jnp.float32) + fyv[pl.ds(f0 + sg, L)]
                xfc = jnp.clip(xf, -1.0, float(W))
                yfc = jnp.clip(yf, -1.0, float(H))
                xt = xfc.astype(jnp.int32)
                yt = yfc.astype(jnp.int32)
                x0i = jnp.where(xt.astype(jnp.float32) > xfc, xt - 1, xt)
                y0i = jnp.where(yt.astype(jnp.float32) > yfc, yt - 1, yt)
                a = xfc - x0i.astype(jnp.float32)
                b = yfc - y0i.astype(jnp.float32)
                x0 = jnp.clip(x0i, 0, W - 1)
                x1 = jnp.clip(x0i + 1, 0, W - 1)
                y0 = jnp.clip(y0i, 0, H - 1)
                y1 = jnp.clip(y0i + 1, 0, H - 1)
                r0 = boff + y0 * W
                r1 = boff + y1 * W
                idxr[0, sl] = r0 + x0
                idxr[1, sl] = r0 + x1
                idxr[2, sl] = r1 + x0
                idxr[3, sl] = r1 + x1
                ia = 1.0 - a
                ib = 1.0 - b
                wgtr[0, sl] = ia * ib
                wgtr[1, sl] = a * ib
                wgtr[2, sl] = ia * b
                wgtr[3, sl] = a * b

        def issue_gathers(slot, ci):
            del ci
            for j in range(4):
                pltpu.async_copy(table.at[idxs[slot].at[j]], rs[slot].at[j],
                                 semg[slot])

        def drain_gathers(slot):
            for j in range(4):
                pltpu.make_async_copy(table.at[idxs[slot].at[j]],
                                      rs[slot].at[j], semg[slot]).wait()

        def blend(slot):
            rr = rs[slot]
            wgtr = wgts[slot]
            outr = outs[slot]
            hi_mask = jnp.full((L,), -65536, jnp.int32)
            for g in range(GPC):
                sl16 = pl.ds(g * L, L)
                pix = lanes + g * L
                wv0 = wgtr[0, sl16]
                wv1 = wgtr[1, sl16]
                wv2 = wgtr[2, sl16]
                wv3 = wgtr[3, sl16]
                jidx = [jnp.full((L,), j, jnp.int32) for j in range(4)]

                @plsc.parallel_loop(0, CH, unroll=4)
                def _(w):
                    ws = lax.broadcast(w, (L,))
                    v = [plsc.load_gather(rr, [jidx[j], pix, ws])
                         for j in range(4)]
                    lo = [plsc.bitcast(vv << 16, jnp.float32) for vv in v]
                    hi = [plsc.bitcast(vv & hi_mask, jnp.float32)
                          for vv in v]
                    lov = (wv0 * lo[0] + wv1 * lo[1]
                           + wv2 * lo[2] + wv3 * lo[3])
                    hiv = (wv0 * hi[0] + wv1 * hi[1]
                           + wv2 * hi[2] + wv3 * hi[3])
                    pc = plsc.pack(lov, hiv,
                                   format=plsc.PackFormat.INTERLEAVED)
                    outr[w, sl16] = plsc.bitcast(pc, jnp.int32)

        def out_slice(ci):
            p0 = base + ci * K
            bb = p0 >> LOG_HW
            yx0 = pl.multiple_of(p0 - (bb << LOG_HW), K)
            return out_hbm.at[bb, :, pl.ds(yx0, K)]

        load_flow(0)
        for ci in range(SLOTS):
            phase1(ci, idxs[ci], wgts[ci])
            issue_gathers(ci, ci)

        def step(t, carry):
            @pl.when(t == RELOAD_T)
            def _():
                load_flow(1)

            for off in range(SLOTS):
                ci = SLOTS * t + off
                drain_gathers(off)

                @pl.when(t >= 1)
                def _():
                    pltpu.make_async_copy(outs[off], out_slice(ci - SLOTS),
                                          semw[off]).wait()

                blend(off)
                pltpu.async_copy(outs[off], out_slice(ci), semw[off])

                @pl.when(t < T - 1)
                def _():
                    phase1(ci + SLOTS, idxs[off], wgts[off])
                    issue_gathers(off, ci + SLOTS)
            return carry

        lax.fori_loop(0, T, step, 0)
        for off in range(SLOTS):
            pltpu.make_async_copy(outs[off], out_slice(NCHUNK - SLOTS + off),
                                  semw[off]).wait()

    return warp


def kernel(input1, input2):
    B, C, H, W = input1.shape

    def rne16(v):
        u = lax.bitcast_convert_type(v, jnp.int32)
        r = u + jnp.int32(0x7FFF) + (lax.shift_right_logical(u, 16) & 1)
        return lax.shift_right_logical(r, 16)

    words = rne16(input1[:, 0::2]) | (rne16(input1[:, 1::2]) << 16)
    table = words.transpose(0, 2, 3, 1).reshape(B * H * W, C // 2)
    flow = input2.reshape(B * 2, H * W)
    ow = _build_warp(B, C, H, W)(table, flow)
    lo = lax.bitcast_convert_type(ow << 16, jnp.float32)
    hi = lax.bitcast_convert_type(ow & jnp.int32(-65536), jnp.float32)
    return jnp.stack([lo, hi], axis=2).reshape(B, C, H, W)


# speedup vs baseline: 1.5594x; 1.5594x over previous
import functools

import jax
import jax.numpy as jnp
from jax import lax
from jax.experimental import pallas as pl
from jax.experimental.pallas import tpu as pltpu
from jax.experimental.pallas import tpu_sc as plsc


@functools.lru_cache(maxsize=None)
def _build_warp(B, C, H, W):
    HW = H * W
    N = B * HW
    CH = C // 2
    info = plsc.get_sparse_core_info()
    NC = info.num_cores
    NW = NC * info.num_subcores
    L = info.num_lanes
    K = 64
    SLOTS = 4
    assert N % NW == 0 and C % (2 * L) == 0
    PPW = N // NW
    assert PPW % (K * SLOTS) == 0
    NCHUNK = PPW // K
    T = NCHUNK // SLOTS
    HALF = NCHUNK // 2
    RELOAD_T = (HALF - SLOTS) // SLOTS
    PPW2 = PPW // 2
    GPC = K // L
    LOG_HW = HW.bit_length() - 1
    assert (1 << LOG_HW) == HW and (1 << (W.bit_length() - 1)) == W

    mesh = plsc.VectorSubcoreMesh(core_axis_name="core", subcore_axis_name="sub")

    scratch = (
        [pltpu.VMEM((PPW2,), jnp.float32)] * 2
        + [pltpu.VMEM((2, 2 * K), jnp.int32)] * SLOTS
        + [pltpu.VMEM((4, K), jnp.float32)] * SLOTS
        + [pltpu.VMEM((2, 2 * K, CH), jnp.int32)] * SLOTS
        + [pltpu.VMEM((C, K), jnp.float32)] * SLOTS
        + [pltpu.SemaphoreType.DMA] * (2 * SLOTS)
    )

    @functools.partial(
        pl.kernel,
        out_type=jax.ShapeDtypeStruct((B, C, HW), jnp.float32),
        mesh=mesh,
        compiler_params=pltpu.CompilerParams(
            use_tc_tiling_on_sc=False, needs_layout_passes=False),
        scratch_types=scratch,
    )
    def warp(table, flow_hbm, out_hbm, *s):
        fxv, fyv = s[0], s[1]
        idxs = s[2:2 + SLOTS]
        wgts = s[2 + SLOTS:2 + 2 * SLOTS]
        rs = s[2 + 2 * SLOTS:2 + 3 * SLOTS]
        outs = s[2 + 3 * SLOTS:2 + 4 * SLOTS]
        semg = s[2 + 4 * SLOTS:2 + 5 * SLOTS]
        semw = s[2 + 5 * SLOTS:2 + 6 * SLOTS]
        wid = lax.axis_index("sub") * NC + lax.axis_index("core")
        base = wid * PPW
        lanes = lax.iota(jnp.int32, L)

        wb = base >> LOG_HW
        wyx = pl.multiple_of(base - (wb << LOG_HW), K)

        def load_flow(halfsel):
            off = pl.multiple_of(wyx + halfsel * PPW2, K)
            pltpu.sync_copy(flow_hbm.at[2 * wb, pl.ds(off, PPW2)], fxv)
            pltpu.sync_copy(flow_hbm.at[2 * wb + 1, pl.ds(off, PPW2)], fyv)

        def phase1(ci, idxr, wgtr):
            p0 = base + ci * K
            f0 = (ci & (HALF - 1)) * K
            for g in range(GPC):
                sg = g * L
                sl = pl.ds(sg, L)
                p = p0 + sg + lanes
                x = p & (W - 1)
                y = (p >> (W.bit_length() - 1)) & (H - 1)
                boff = p - (p & (HW - 1))
                xf = x.astype(jnp.float32) + fxv[pl.ds(f0 + sg, L)]
                yf = y.astype(jnp.float32) + fyv[pl.ds(f0 + sg, L)]
                xfc = jnp.clip(xf, -1.0, float(W))
                yfc = jnp.clip(yf, -1.0, float(H))
                xt = xfc.astype(jnp.int32)
                yt = yfc.astype(jnp.int32)
                x0i = jnp.where(xt.astype(jnp.float32) > xfc, xt - 1, xt)
                y0i = jnp.where(yt.astype(jnp.float32) > yfc, yt - 1, yt)
                a = xfc - x0i.astype(jnp.float32)
                b = yfc - y0i.astype(jnp.float32)
                x0 = jnp.clip(x0i, 0, W - 1)
                x1 = jnp.clip(x0i + 1, 0, W - 1)
                y0 = jnp.clip(y0i, 0, H - 1)
                y1 = jnp.clip(y0i + 1, 0, H - 1)
                r0 = boff + y0 * W
                r1 = boff + y1 * W
                idxr[0, sl] = r0 + x0
                idxr[0, pl.ds(K + sg, L)] = r0 + x1
                idxr[1, sl] = r1 + x0
                idxr[1, pl.ds(K + sg, L)] = r1 + x1
                ia = 1.0 - a
                ib = 1.0 - b
                wgtr[0, sl] = ia * ib
                wgtr[1, sl] = a * ib
                wgtr[2, sl] = ia * b
                wgtr[3, sl] = a * b

        def issue_gathers(slot, ci):
            del ci
            for j in range(2):
                pltpu.async_copy(table.at[idxs[slot].at[j]], rs[slot].at[j],
                                 semg[slot])

        def drain_gathers(slot):
            for j in range(2):
                pltpu.make_async_copy(table.at[idxs[slot].at[j]],
                                      rs[slot].at[j], semg[slot]).wait()

        def blend(slot):
            rr = rs[slot]
            wgtr = wgts[slot]
            outr = outs[slot]
            hi_mask = jnp.full((L,), -65536, jnp.int32)
            for g in range(GPC):
                sl16 = pl.ds(g * L, L)
                pix = lanes + g * L
                wv0 = wgtr[0, sl16]
                wv1 = wgtr[1, sl16]
                wv2 = wgtr[2, sl16]
                wv3 = wgtr[3, sl16]
                jidx = [jnp.full((L,), j, jnp.int32) for j in (0, 0, 1, 1)]
                pixo = [pix, pix + K, pix, pix + K]

                @plsc.parallel_loop(0, CH, unroll=4)
                def _(w):
                    ws = lax.broadcast(w, (L,))
                    v = [plsc.load_gather(rr, [jidx[j], pixo[j], ws])
                         for j in range(4)]
                    lo = [plsc.bitcast(vv << 16, jnp.float32) for vv in v]
                    hi = [plsc.bitcast(vv & hi_mask, jnp.float32)
                          for vv in v]
                    lov = (wv0 * lo[0] + wv1 * lo[1]
                           + wv2 * lo[2] + wv3 * lo[3])
                    hiv = (wv0 * hi[0] + wv1 * hi[1]
                           + wv2 * hi[2] + wv3 * hi[3])
                    outr[2 * w, sl16] = lov
                    outr[2 * w + 1, sl16] = hiv

        def out_slice(ci):
            p0 = base + ci * K
            bb = p0 >> LOG_HW
            yx0 = pl.multiple_of(p0 - (bb << LOG_HW), K)
            return out_hbm.at[bb, :, pl.ds(yx0, K)]

        load_flow(0)
        for ci in range(SLOTS):
            phase1(ci, idxs[ci], wgts[ci])
            issue_gathers(ci, ci)

        def step(t, carry):
            @pl.when(t == RELOAD_T)
            def _():
                load_flow(1)

            for off in range(SLOTS):
                ci = SLOTS * t + off
                drain_gathers(off)

                @pl.when(t >= 1)
                def _():
                    pltpu.make_async_copy(outs[off], out_slice(ci - SLOTS),
                                          semw[off]).wait()

                blend(off)
                pltpu.async_copy(outs[off], out_slice(ci), semw[off])

                @pl.when(t < T - 1)
                def _():
                    phase1(ci + SLOTS, idxs[off], wgts[off])
                    issue_gathers(off, ci + SLOTS)
            return carry

        lax.fori_loop(0, T, step, 0)
        for off in range(SLOTS):
            pltpu.make_async_copy(outs[off], out_slice(NCHUNK - SLOTS + off),
                                  semw[off]).wait()

    return warp


def kernel(input1, input2):
    B, C, H, W = input1.shape

    def rne16(v):
        u = lax.bitcast_convert_type(v, jnp.int32)
        r = u + jnp.int32(0x7FFF) + (lax.shift_right_logical(u, 16) & 1)
        return lax.shift_right_logical(r, 16)

    words = rne16(input1[:, 0::2]) | (rne16(input1[:, 1::2]) << 16)
    table = words.transpose(0, 2, 3, 1).reshape(B * H * W, C // 2)
    flow = input2.reshape(B * 2, H * W)
    out = _build_warp(B, C, H, W)(table, flow)
    return out.reshape(B, C, H, W)
